# Initial kernel scaffold; baseline (speedup 1.0000x reference)
import jax
import jax.numpy as jnp
from jax.experimental import pallas as pl


def kernel(graph, node_features, train_pos_samples, W1, b1, W2, b2, Wp1, bp1, Wp2, bp2):
    raise NotImplementedError("write your pallas kernel here")


# speedup vs baseline: 7.9190x; 7.9190x over previous
import functools

import jax
import jax.numpy as jnp
from jax import lax
from jax.experimental import pallas as pl
from jax.experimental.pallas import tpu as pltpu
from jax.experimental.pallas import tpu_sc as plsc

N = 10000
E = 320000
D = 128
B = 4096
L = 3
NC = 2
NS = 16
NW = NC * NS
CHUNK = 128
NCH = 80
E_PAD = NW * NCH * CHUNK
RPT = 640
N_ACC = NS * RPT

_MESH = plsc.VectorSubcoreMesh(core_axis_name="c", subcore_axis_name="s")


def _make_edge_pass(compute_deg):
  out_type = [jax.ShapeDtypeStruct((NC, N_ACC, D), jnp.float32)]
  if compute_deg:
    out_type.append(jax.ShapeDtypeStruct((NC * N_ACC,), jnp.float32))
  scratch = [
      pltpu.VMEM((NCH, CHUNK), jnp.int32),
      pltpu.VMEM((NCH, CHUNK), jnp.int32),
      pltpu.VMEM((CHUNK, D), jnp.float32),
      pltpu.VMEM((CHUNK,), jnp.float32),
      pltpu.VMEM((RPT,), jnp.float32),
      pltpu.VMEM_SHARED((N_ACC, D), jnp.float32),
      pltpu.VMEM_SHARED((N_ACC,), jnp.float32),
      pltpu.SemaphoreType.DMA,
  ]

  @functools.partial(
      pl.kernel,
      out_type=tuple(out_type) if compute_deg else out_type[0],
      mesh=_MESH,
      scratch_types=scratch,
  )
  def edge_pass(src3, dst3, x, zrows, *rest):
    if compute_deg:
      (acc_out, deg_out, src_v, dst_v, rows_v, ones_v, deg_v, acc_sh,
       deg_sh, sem) = rest
    else:
      (acc_out, src_v, dst_v, rows_v, ones_v, deg_v, acc_sh,
       deg_sh, sem) = rest
    cid = lax.axis_index("c")
    sid = lax.axis_index("s")
    wid = cid * NS + sid

    pltpu.sync_copy(zrows, acc_sh.at[pl.ds(sid * RPT, RPT)])
    if compute_deg:
      for j in range(RPT // 16):
        deg_v[pl.ds(j * 16, 16)] = jnp.zeros((16,), jnp.float32)
      pltpu.sync_copy(deg_v, deg_sh.at[pl.ds(sid * RPT, RPT)])
      for j in range(CHUNK // 16):
        ones_v[pl.ds(j * 16, 16)] = jnp.ones((16,), jnp.float32)
    pltpu.sync_copy(src3.at[wid], src_v)
    pltpu.sync_copy(dst3.at[wid], dst_v)
    plsc.subcore_barrier()

    def body(i, carry):
      pltpu.async_copy(x.at[src_v.at[i]], rows_v, sem).wait()
      pltpu.sync_copy(rows_v, acc_sh.at[dst_v.at[i]], add=True)
      if compute_deg:
        pltpu.sync_copy(ones_v, deg_sh.at[dst_v.at[i]], add=True)
      return carry

    lax.fori_loop(0, NCH, body, 0)
    plsc.subcore_barrier()

    pltpu.sync_copy(acc_sh.at[pl.ds(sid * RPT, RPT)],
                    acc_out.at[cid, pl.ds(sid * RPT, RPT)])
    if compute_deg:
      pltpu.sync_copy(deg_sh.at[pl.ds(sid * RPT, RPT)], deg_v)
      pltpu.sync_copy(deg_v, deg_out.at[pl.ds(cid * N_ACC + sid * RPT, RPT)])

  return edge_pass


_edge_pass_deg = _make_edge_pass(True)
_edge_pass_nodeg = _make_edge_pass(False)

_GPT = (B * L) // NW


@functools.partial(
    pl.kernel,
    out_type=jax.ShapeDtypeStruct((B * L, D), jnp.float32),
    mesh=_MESH,
    scratch_types=[
        pltpu.VMEM((_GPT // CHUNK, CHUNK), jnp.int32),
        pltpu.VMEM((CHUNK, D), jnp.float32),
        pltpu.SemaphoreType.DMA,
    ],
)
def _tuple_gather(idx3, h, out, idx_v, rows_v, sem):
  cid = lax.axis_index("c")
  sid = lax.axis_index("s")
  wid = cid * NS + sid
  pltpu.sync_copy(idx3.at[wid], idx_v)
  for j in range(_GPT // CHUNK):
    pltpu.async_copy(h.at[idx_v.at[j]], rows_v, sem).wait()
    pltpu.sync_copy(rows_v, out.at[pl.ds(wid * _GPT + j * CHUNK, CHUNK)])


def _layer_body(acc_ref, x_ref, deg_ref, w_ref, b_ref, out_ref, *, relu):
  deg = deg_ref[0] + deg_ref[1]
  r = 1.0 / (deg + 1.0)
  h = (acc_ref[0] + acc_ref[1] + x_ref[...]) * r
  y = jnp.dot(h, w_ref[...], preferred_element_type=jnp.float32) + b_ref[...]
  out_ref[...] = jnp.maximum(y, 0.0) if relu else y


def _sage_layer(acc, x, deg3, w, b, relu):
  R = 1000
  return pl.pallas_call(
      functools.partial(_layer_body, relu=relu),
      grid=(N // R,),
      in_specs=[
          pl.BlockSpec((NC, R, D), lambda i: (0, i, 0)),
          pl.BlockSpec((R, D), lambda i: (i, 0)),
          pl.BlockSpec((NC, R, 1), lambda i: (0, i, 0)),
          pl.BlockSpec((D, D), lambda i: (0, 0)),
          pl.BlockSpec((1, D), lambda i: (0, 0)),
      ],
      out_specs=pl.BlockSpec((R, D), lambda i: (i, 0)),
      out_shape=jax.ShapeDtypeStruct((N, D), jnp.float32),
  )(acc, x, deg3, w, b)


def _pred_body(g_ref, w1_ref, b1_ref, w2_ref, b2_ref, out_ref):
  t = jnp.dot(g_ref[...], w1_ref[...], preferred_element_type=jnp.float32)
  t = jnp.maximum(t + b1_ref[...], 0.0)
  out_ref[...] = (
      jnp.dot(t, w2_ref[...], preferred_element_type=jnp.float32) + b2_ref[...]
  )


def _predictor(g, wp1, bp1, wp2, bp2):
  R = 512
  return pl.pallas_call(
      _pred_body,
      grid=(B // R,),
      in_specs=[
          pl.BlockSpec((R, L * D), lambda i: (i, 0)),
          pl.BlockSpec((L * D, 32), lambda i: (0, 0)),
          pl.BlockSpec((1, 32), lambda i: (0, 0)),
          pl.BlockSpec((32, 1000), lambda i: (0, 0)),
          pl.BlockSpec((1, 1000), lambda i: (0, 0)),
      ],
      out_specs=pl.BlockSpec((R, 1000), lambda i: (i, 0)),
      out_shape=jax.ShapeDtypeStruct((B, 1000), jnp.float32),
  )(g, wp1, bp1, wp2, bp2)


def kernel(graph, node_features, train_pos_samples, W1, b1, W2, b2,
           Wp1, bp1, Wp2, bp2):
  src = graph[0].astype(jnp.int32)
  dst = graph[1].astype(jnp.int32)
  npad = E_PAD - E
  pad = jnp.arange(npad, dtype=jnp.int32)
  src_p = jnp.concatenate([src, pad % N])
  dst_p = jnp.concatenate([dst, N + pad % (N_ACC - N)])
  src3 = src_p.reshape(NW, NCH, CHUNK)
  dst3 = dst_p.reshape(NW, NCH, CHUNK)
  zrows = jnp.zeros((RPT, D), jnp.float32)

  x = node_features
  acc1, deg = _edge_pass_deg(src3, dst3, x, zrows)
  deg3 = deg.reshape(NC, N_ACC)[:, :N, None]
  h1 = _sage_layer(acc1[:, :N], x, deg3, W1, b1.reshape(1, D), True)
  acc2 = _edge_pass_nodeg(src3, dst3, h1, zrows)
  h2 = _sage_layer(acc2[:, :N], h1, deg3, W2, b2.reshape(1, D), False)

  idx3 = train_pos_samples.astype(jnp.int32).reshape(NW, _GPT // CHUNK, CHUNK)
  g = _tuple_gather(idx3, h2).reshape(B, L * D)
  return _predictor(g, Wp1, bp1.reshape(1, 32), Wp2, bp2.reshape(1, 1000))


# speedup vs baseline: 11.5204x; 1.4548x over previous
import functools

import jax
import jax.numpy as jnp
from jax import lax
from jax.experimental import pallas as pl
from jax.experimental.pallas import tpu as pltpu
from jax.experimental.pallas import tpu_sc as plsc

N = 10000
E = 320000
D = 128
B = 4096
L = 3
NC = 2
NS = 16
NW = NC * NS
CHUNK = 128
NCH = 80
E_PAD = NW * NCH * CHUNK
RPT = 640
N_ACC = NS * RPT
NBUF = 2

_MESH = plsc.VectorSubcoreMesh(core_axis_name="c", subcore_axis_name="s")


def _make_edge_pass(compute_deg):
  out_type = [jax.ShapeDtypeStruct((NC, N_ACC, D), jnp.float32)]
  if compute_deg:
    out_type.append(jax.ShapeDtypeStruct((NC * N_ACC,), jnp.float32))
  scratch = [
      pltpu.VMEM((NBUF, CHUNK), jnp.int32),
      pltpu.VMEM((NCH, CHUNK), jnp.int32),
      pltpu.VMEM((NBUF, CHUNK, D), jnp.float32),
      pltpu.VMEM((CHUNK,), jnp.float32),
      pltpu.VMEM((RPT,), jnp.float32),
      pltpu.VMEM_SHARED((N_ACC, D), jnp.float32),
      pltpu.VMEM_SHARED((N_ACC,), jnp.float32),
  ] + [pltpu.SemaphoreType.DMA] * (2 * NBUF + 1)

  @functools.partial(
      pl.kernel,
      out_type=tuple(out_type) if compute_deg else out_type[0],
      mesh=_MESH,
      scratch_types=scratch,
  )
  def edge_pass(src_flat, dst3, x, zrows, *rest):
    if compute_deg:
      (acc_out, deg_out, sidx_v, dst_v, rows_v, ones_v, deg_v, acc_sh,
       deg_sh, *sems) = rest
    else:
      (acc_out, sidx_v, dst_v, rows_v, ones_v, deg_v, acc_sh,
       deg_sh, *sems) = rest
    gsems, isems, dsem = sems[:NBUF], sems[NBUF:2 * NBUF], sems[2 * NBUF]
    cid = lax.axis_index("c")
    sid = lax.axis_index("s")
    wid = cid * NS + sid

    pltpu.sync_copy(zrows, acc_sh.at[pl.ds(sid * RPT, RPT)])
    if compute_deg:
      for j in range(RPT // 16):
        deg_v[pl.ds(j * 16, 16)] = jnp.zeros((16,), jnp.float32)
      pltpu.sync_copy(deg_v, deg_sh.at[pl.ds(sid * RPT, RPT)])
      for j in range(CHUNK // 16):
        ones_v[pl.ds(j * 16, 16)] = jnp.ones((16,), jnp.float32)
    pltpu.sync_copy(dst3.at[wid], dst_v)
    plsc.subcore_barrier()

    def src_chunk(i):
      return src_flat.at[pl.ds((wid * NCH + i) * CHUNK, CHUNK)]

    for b in range(NBUF):
      pltpu.sync_copy(src_chunk(b), sidx_v.at[b])
      pltpu.async_copy(x.at[sidx_v.at[b]], rows_v.at[b], gsems[b])

    def body(g, carry):
      for b in range(NBUF):
        i = g * NBUF + b
        nxt = i + NBUF
        pltpu.make_async_copy(x.at[sidx_v.at[b]], rows_v.at[b],
                              gsems[b]).wait()

        @pl.when(nxt < NCH)
        def _():
          pltpu.async_copy(src_chunk(nxt), sidx_v.at[b], isems[b])

        pltpu.sync_copy(rows_v.at[b], acc_sh.at[dst_v.at[i]], add=True)
        if compute_deg:
          pltpu.async_copy(ones_v, deg_sh.at[dst_v.at[i]], dsem, add=True)

        @pl.when(nxt < NCH)
        def _():
          pltpu.make_async_copy(src_chunk(nxt), sidx_v.at[b],
                                isems[b]).wait()
          pltpu.async_copy(x.at[sidx_v.at[b]], rows_v.at[b], gsems[b])
      return carry

    lax.fori_loop(0, NCH // NBUF, body, 0)
    if compute_deg:
      def drain(i, carry):
        pltpu.make_async_copy(ones_v, deg_sh.at[dst_v.at[i]], dsem).wait()
        return carry

      lax.fori_loop(0, NCH, drain, 0)
    plsc.subcore_barrier()

    pltpu.sync_copy(acc_sh.at[pl.ds(sid * RPT, RPT)],
                    acc_out.at[cid, pl.ds(sid * RPT, RPT)])
    if compute_deg:
      pltpu.sync_copy(deg_sh.at[pl.ds(sid * RPT, RPT)], deg_v)
      pltpu.sync_copy(deg_v, deg_out.at[pl.ds(cid * N_ACC + sid * RPT, RPT)])

  return edge_pass


_edge_pass_deg = _make_edge_pass(True)
_edge_pass_nodeg = _make_edge_pass(False)

_GPT = (B * L) // NW


@functools.partial(
    pl.kernel,
    out_type=jax.ShapeDtypeStruct((B * L, D), jnp.float32),
    mesh=_MESH,
    scratch_types=[
        pltpu.VMEM((_GPT // CHUNK, CHUNK), jnp.int32),
        pltpu.VMEM((CHUNK, D), jnp.float32),
        pltpu.SemaphoreType.DMA,
    ],
)
def _tuple_gather(idx3, h, out, idx_v, rows_v, sem):
  cid = lax.axis_index("c")
  sid = lax.axis_index("s")
  wid = cid * NS + sid
  pltpu.sync_copy(idx3.at[wid], idx_v)
  for j in range(_GPT // CHUNK):
    pltpu.async_copy(h.at[idx_v.at[j]], rows_v, sem).wait()
    pltpu.sync_copy(rows_v, out.at[pl.ds(wid * _GPT + j * CHUNK, CHUNK)])


def _layer_body(acc_ref, x_ref, deg_ref, w_ref, b_ref, out_ref, *, relu):
  deg = deg_ref[0] + deg_ref[1]
  r = 1.0 / (deg + 1.0)
  h = (acc_ref[0] + acc_ref[1] + x_ref[...]) * r
  y = jnp.dot(h, w_ref[...], preferred_element_type=jnp.float32) + b_ref[...]
  out_ref[...] = jnp.maximum(y, 0.0) if relu else y


def _sage_layer(acc, x, deg3, w, b, relu):
  R = 1000
  return pl.pallas_call(
      functools.partial(_layer_body, relu=relu),
      grid=(N // R,),
      in_specs=[
          pl.BlockSpec((NC, R, D), lambda i: (0, i, 0)),
          pl.BlockSpec((R, D), lambda i: (i, 0)),
          pl.BlockSpec((NC, R, 1), lambda i: (0, i, 0)),
          pl.BlockSpec((D, D), lambda i: (0, 0)),
          pl.BlockSpec((1, D), lambda i: (0, 0)),
      ],
      out_specs=pl.BlockSpec((R, D), lambda i: (i, 0)),
      out_shape=jax.ShapeDtypeStruct((N, D), jnp.float32),
  )(acc, x, deg3, w, b)


def _pred_body(g_ref, w1_ref, b1_ref, w2_ref, b2_ref, out_ref):
  t = jnp.dot(g_ref[...], w1_ref[...], preferred_element_type=jnp.float32)
  t = jnp.maximum(t + b1_ref[...], 0.0)
  out_ref[...] = (
      jnp.dot(t, w2_ref[...], preferred_element_type=jnp.float32) + b2_ref[...]
  )


def _predictor(g, wp1, bp1, wp2, bp2):
  R = 512
  return pl.pallas_call(
      _pred_body,
      grid=(B // R,),
      in_specs=[
          pl.BlockSpec((R, L * D), lambda i: (i, 0)),
          pl.BlockSpec((L * D, 32), lambda i: (0, 0)),
          pl.BlockSpec((1, 32), lambda i: (0, 0)),
          pl.BlockSpec((32, 1000), lambda i: (0, 0)),
          pl.BlockSpec((1, 1000), lambda i: (0, 0)),
      ],
      out_specs=pl.BlockSpec((R, 1000), lambda i: (i, 0)),
      out_shape=jax.ShapeDtypeStruct((B, 1000), jnp.float32),
  )(g, wp1, bp1, wp2, bp2)


def kernel(graph, node_features, train_pos_samples, W1, b1, W2, b2,
           Wp1, bp1, Wp2, bp2):
  src = graph[0].astype(jnp.int32)
  dst = graph[1].astype(jnp.int32)
  npad = E_PAD - E
  pad = jnp.arange(npad, dtype=jnp.int32)
  src_p = jnp.concatenate([src, pad % N])
  dst3 = jnp.concatenate([dst, N + pad % (N_ACC - N)]).reshape(NW, NCH, CHUNK)
  zrows = jnp.zeros((RPT, D), jnp.float32)

  x = node_features
  acc1, deg = _edge_pass_deg(src_p, dst3, x, zrows)
  deg3 = deg.reshape(NC, N_ACC, 1)
  h1 = _sage_layer(acc1, x, deg3, W1, b1.reshape(1, D), True)
  acc2 = _edge_pass_nodeg(src_p, dst3, h1, zrows)
  h2 = _sage_layer(acc2, h1, deg3, W2, b2.reshape(1, D), False)

  idx3 = train_pos_samples.astype(jnp.int32).reshape(NW, _GPT // CHUNK, CHUNK)
  g = _tuple_gather(idx3, h2).reshape(B, L * D)
  return _predictor(g, Wp1, bp1.reshape(1, 32), Wp2, bp2.reshape(1, 1000))
